# Initial kernel scaffold; baseline (speedup 1.0000x reference)
import jax
import jax.numpy as jnp
from jax.experimental import pallas as pl


def kernel(q_pts, s_pts, s_feats, neighb_inds, W_d1, g_d1, b_d1, W_d2, b_d2, g_g0, b_g0, W_g, b_g, g_a0, b_a0, W_a1, g_a1, b_a1, W_a2, b_a2):
    raise NotImplementedError("write your pallas kernel here")


# speedup vs baseline: 1.7688x; 1.7688x over previous
import functools

import jax
import jax.numpy as jnp
from jax import lax
from jax.experimental import pallas as pl
from jax.experimental.pallas import tpu as pltpu
from jax.experimental.pallas import tpu_sc as plsc

_NW = 32


def _sc_gather(feats, pts_lin, idx_flat):
    rows = idx_flat.shape[0]
    c = feats.shape[1]
    npts4 = pts_lin.shape[0]
    rpw = rows // _NW
    ch = 80 if rpw % 80 == 0 else rpw
    iters = rpw // ch
    nv16 = ch * 4 // 16
    mesh = plsc.VectorSubcoreMesh(core_axis_name="c", subcore_axis_name="s")

    @functools.partial(
        pl.kernel,
        mesh=mesh,
        compiler_params=pltpu.CompilerParams(needs_layout_passes=False),
        out_type=[
            jax.ShapeDtypeStruct((rows, c), jnp.float32),
            jax.ShapeDtypeStruct((rows * 4,), jnp.float32),
        ],
        scratch_types=[
            pltpu.VMEM((ch,), jnp.int32),
            pltpu.VMEM((ch, c), jnp.float32),
            pltpu.VMEM((npts4,), jnp.float32),
            pltpu.VMEM((ch * 4,), jnp.float32),
            pltpu.SemaphoreType.DMA,
        ],
    )
    def gk(idx_hbm, feats_hbm, ptslin_hbm, outf_hbm, outp_hbm,
           idx_v, rows_v, ptab_v, pbuf_v, sem1):
        wid = lax.axis_index("s") * 2 + lax.axis_index("c")
        base = wid * rpw
        pltpu.sync_copy(ptslin_hbm, ptab_v)
        io4 = lax.iota(jnp.int32, 16) // 4
        ioc = lax.iota(jnp.int32, 16) % 4

        def body(j, carry):
            off = base + j * ch
            pltpu.sync_copy(idx_hbm.at[pl.ds(off, ch)], idx_v)
            cp1 = pltpu.async_copy(feats_hbm.at[idx_v], rows_v, sem1)

            def inner(t, carry2):
                e0 = t * 16
                rvec = e0 // 4 + io4
                idxg = plsc.load_gather(idx_v, [rvec])
                vals = plsc.load_gather(ptab_v, [idxg * 4 + ioc])
                pbuf_v[pl.ds(e0, 16)] = vals
                return carry2

            lax.fori_loop(0, nv16, inner, 0)
            cp1.wait()
            pltpu.sync_copy(rows_v, outf_hbm.at[pl.ds(off, ch)])
            pltpu.sync_copy(pbuf_v, outp_hbm.at[pl.ds(off * 4, ch * 4)])
            return carry

        lax.fori_loop(0, iters, body, 0)

    outf, outp = gk(idx_flat, feats, pts_lin)
    return outf, outp.reshape(rows, 4)


def _lrelu(x):
    return jnp.where(x >= 0, x, 0.1 * x)


def _rep_points(v, bp, h):
    k = v.shape[1]
    return jnp.broadcast_to(v[:, None, :], (bp, h, k)).reshape(bp * h, k)


def _p0_body(spg_ref, qp_ref, out_ref):
    b = pl.program_id(0)
    br = spg_ref.shape[0]
    bp = qp_ref.shape[0]
    h = br // bp
    d3 = spg_ref[:, :3] - _rep_points(qp_ref[...], bp, h)
    d4 = jnp.concatenate([d3, jnp.ones((br, 1), jnp.float32)], axis=1)
    m = lax.dot_general(d4, d4, (((0,), (0,)), ((), ())),
                        preferred_element_type=jnp.float32)

    @pl.when(b == 0)
    def _():
        out_ref[...] = m

    @pl.when(b > 0)
    def _():
        out_ref[...] += m


def _compute_ge(spg_ref, qp_ref, wd1_ref, wd2_ref, bd2_ref, s1_ref, t1_ref):
    br = spg_ref.shape[0]
    bp = qp_ref.shape[0]
    h = br // bp
    d3 = spg_ref[:, :3] - _rep_points(qp_ref[...], bp, h)
    y1 = jnp.dot(d3, wd1_ref[...], preferred_element_type=jnp.float32)
    a1 = _lrelu(y1 * s1_ref[...] + t1_ref[...])
    return jnp.dot(a1, wd2_ref[...], preferred_element_type=jnp.float32) + bd2_ref[...]


def _pa_body(nv_ref, spg_ref, qp_ref, wd1_ref, wd2_ref, bd2_ref, s1_ref, t1_ref,
             sge_ref, sge2_ref, nf0_ref, sums_ref):
    b = pl.program_id(0)
    br = nv_ref.shape[0]
    bp = qp_ref.shape[0]
    h = br // bp
    c = nv_ref.shape[1]
    ge = _compute_ge(spg_ref, qp_ref, wd1_ref, wd2_ref, bd2_ref, s1_ref, t1_ref)
    nvf = nv_ref[...] - ge
    ge3 = ge.reshape(bp, h, c)
    sge_ref[...] = jnp.sum(ge3, axis=1)
    sge2_ref[...] = jnp.sum(ge3 * ge3, axis=1)
    nf0_ref[...] = nvf.reshape(bp, h, c)[:, 0, :]
    sq = jnp.concatenate([jnp.sum(nvf, axis=0)[None, :],
                          jnp.sum(nvf * nvf, axis=0)[None, :]], axis=0)

    @pl.when(b == 0)
    def _():
        sums_ref[...] = sq

    @pl.when(b > 0)
    def _():
        sums_ref[...] += sq


def _q0_block(nf0_ref, wg_ref, bg_ref, s2_ref, t2_ref):
    a = _lrelu(nf0_ref[...] * s2_ref[...] + t2_ref[...])
    return jnp.dot(a, wg_ref[...], preferred_element_type=jnp.float32) + bg_ref[...]


def _pb1_body(nf0_ref, sge_ref, sge2_ref, wg_ref, bg_ref, s2_ref, t2_ref,
              out_ref, *, h):
    b = pl.program_id(0)
    q0 = _q0_block(nf0_ref, wg_ref, bg_ref, s2_ref, t2_ref)
    sge = sge_ref[...]
    sge2 = sge2_ref[...]
    hf = jnp.float32(h)
    s = jnp.sum(hf * q0 - sge, axis=0)[None, :]
    q = jnp.sum(hf * (q0 * q0) - 2.0 * q0 * sge + sge2, axis=0)[None, :]
    sq = jnp.concatenate([s, q], axis=0)

    @pl.when(b == 0)
    def _():
        out_ref[...] = sq

    @pl.when(b > 0)
    def _():
        out_ref[...] += sq


def _pb2_body(spg_ref, qp_ref, nf0_ref, wd1_ref, wd2_ref, bd2_ref,
              wg_ref, bg_ref, wa1_ref, s1_ref, t1_ref, s2_ref, t2_ref,
              s3_ref, t3_ref, sums_ref):
    b = pl.program_id(0)
    br = spg_ref.shape[0]
    bp = qp_ref.shape[0]
    h = br // bp
    ge = _compute_ge(spg_ref, qp_ref, wd1_ref, wd2_ref, bd2_ref, s1_ref, t1_ref)
    q0 = _q0_block(nf0_ref, wg_ref, bg_ref, s2_ref, t2_ref)
    qf = _rep_points(q0, bp, h) - ge
    a3 = _lrelu(qf * s3_ref[...] + t3_ref[...])
    z4 = jnp.dot(a3, wa1_ref[...], preferred_element_type=jnp.float32)
    sq = jnp.concatenate([jnp.sum(z4, axis=0)[None, :],
                          jnp.sum(z4 * z4, axis=0)[None, :]], axis=0)

    @pl.when(b == 0)
    def _():
        sums_ref[...] = sq

    @pl.when(b > 0)
    def _():
        sums_ref[...] += sq


def _pd_body(nv_ref, spg_ref, qp_ref, nf0_ref, wd1_ref, wd2_ref, bd2_ref,
             wg_ref, bg_ref, wa1_ref, wa2_ref, ba2_ref, emat_ref,
             s1_ref, t1_ref, s2_ref, t2_ref, s3_ref, t3_ref, s4_ref, t4_ref,
             out_ref):
    br = nv_ref.shape[0]
    bp = qp_ref.shape[0]
    h = br // bp
    c = nv_ref.shape[1]
    cpg = wa2_ref.shape[1]
    ge = _compute_ge(spg_ref, qp_ref, wd1_ref, wd2_ref, bd2_ref, s1_ref, t1_ref)
    nvf = nv_ref[...] - ge
    a2 = _lrelu(nvf * s2_ref[...] + t2_ref[...])
    nvf2 = jnp.dot(a2, wg_ref[...], preferred_element_type=jnp.float32) + bg_ref[...]
    q0 = _q0_block(nf0_ref, wg_ref, bg_ref, s2_ref, t2_ref)
    qf = _rep_points(q0, bp, h) - ge
    a3 = _lrelu(qf * s3_ref[...] + t3_ref[...])
    z4 = jnp.dot(a3, wa1_ref[...], preferred_element_type=jnp.float32)
    a4 = _lrelu(z4 * s4_ref[...] + t4_ref[...])
    aw = jnp.dot(a4, wa2_ref[...], preferred_element_type=jnp.float32) + ba2_ref[...]
    r = aw.reshape(bp, h, cpg)
    m = jnp.max(r, axis=1, keepdims=True)
    e = jnp.exp(r - m)
    sm = (e / jnp.sum(e, axis=1, keepdims=True)).reshape(br, cpg)
    smx = jnp.dot(sm, emat_ref[...], preferred_element_type=jnp.float32)
    out_ref[...] = jnp.sum((nvf2 * smx).reshape(bp, h, c), axis=1)


def _bn_scale_shift(sums, count, g, b):
    mean = sums[0] / count
    var = sums[1] / count - mean * mean
    s = g / jnp.sqrt(var + 1e-5)
    return s[None, :], (b - mean * s)[None, :]


def kernel(q_pts, s_pts, s_feats, neighb_inds, W_d1, g_d1, b_d1, W_d2, b_d2,
           g_g0, b_g0, W_g, b_g, g_a0, b_a0, W_a1, g_a1, b_a1, W_a2, b_a2):
    n = q_pts.shape[0]
    h = neighb_inds.shape[1]
    c = s_feats.shape[1]
    cpg = W_a2.shape[1]
    g = c // cpg
    rows = n * h
    pw = 4

    bp = 40 if n % 40 == 0 else 8
    br = bp * h
    nb = rows // br
    rowsf = jnp.float32(rows)

    idx_flat = neighb_inds.reshape(-1).astype(jnp.int32)
    pts_lin = jnp.pad(s_pts, ((0, 0), (0, pw - s_pts.shape[1]))).reshape(-1)

    nv, spg = _sc_gather(s_feats, pts_lin, idx_flat)

    row_spec_c = pl.BlockSpec((br, c), lambda b: (b, 0))
    row_spec_p = pl.BlockSpec((br, pw), lambda b: (b, 0))
    qp_spec = pl.BlockSpec((bp, 3), lambda b: (b, 0))
    pt_spec_c = pl.BlockSpec((bp, c), lambda b: (b, 0))
    full = lambda shape: pl.BlockSpec(shape, lambda b: tuple(0 for _ in shape))
    vec = full((1, c))

    m0 = pl.pallas_call(
        _p0_body,
        grid=(nb,),
        in_specs=[row_spec_p, qp_spec],
        out_specs=full((4, 4)),
        out_shape=jax.ShapeDtypeStruct((4, 4), jnp.float32),
    )(spg, q_pts)

    mean3 = m0[3, :3] / rowsf
    cov = m0[:3, :3] / rowsf - mean3[:, None] * mean3[None, :]
    m1 = mean3 @ W_d1
    var1 = jnp.sum((cov @ W_d1) * W_d1, axis=0)
    s1 = (g_d1 / jnp.sqrt(var1 + 1e-5))[None, :]
    t1 = (b_d1 - m1 * s1[0])[None, :]

    sge, sge2, nf0, sums2 = pl.pallas_call(
        _pa_body,
        grid=(nb,),
        in_specs=[row_spec_c, row_spec_p, qp_spec, full((3, c)), full((c, c)),
                  vec, vec, vec],
        out_specs=[pt_spec_c, pt_spec_c, pt_spec_c, full((2, c))],
        out_shape=[jax.ShapeDtypeStruct((n, c), jnp.float32),
                   jax.ShapeDtypeStruct((n, c), jnp.float32),
                   jax.ShapeDtypeStruct((n, c), jnp.float32),
                   jax.ShapeDtypeStruct((2, c), jnp.float32)],
    )(nv, spg, q_pts, W_d1, W_d2, b_d2[None, :], s1, t1)
    s2, t2 = _bn_scale_shift(sums2, rowsf, g_g0, b_g0)

    sums3 = pl.pallas_call(
        functools.partial(_pb1_body, h=h),
        grid=(nb,),
        in_specs=[pt_spec_c, pt_spec_c, pt_spec_c, full((c, c)), vec, vec, vec],
        out_specs=full((2, c)),
        out_shape=jax.ShapeDtypeStruct((2, c), jnp.float32),
    )(nf0, sge, sge2, W_g, b_g[None, :], s2, t2)
    s3, t3 = _bn_scale_shift(sums3, rowsf, g_a0, b_a0)

    sums4 = pl.pallas_call(
        _pb2_body,
        grid=(nb,),
        in_specs=[row_spec_p, qp_spec, pt_spec_c, full((3, c)), full((c, c)),
                  vec, full((c, c)), vec, full((c, c)),
                  vec, vec, vec, vec, vec, vec],
        out_specs=full((2, c)),
        out_shape=jax.ShapeDtypeStruct((2, c), jnp.float32),
    )(spg, q_pts, nf0, W_d1, W_d2, b_d2[None, :], W_g, b_g[None, :], W_a1,
      s1, t1, s2, t2, s3, t3)
    s4, t4 = _bn_scale_shift(sums4, rowsf, g_a1, b_a1)

    emat = (jnp.arange(c)[None, :] // g == jnp.arange(cpg)[:, None]).astype(jnp.float32)

    out = pl.pallas_call(
        _pd_body,
        grid=(nb,),
        in_specs=[row_spec_c, row_spec_p, qp_spec, pt_spec_c, full((3, c)),
                  full((c, c)), vec, full((c, c)), vec, full((c, c)),
                  full((c, cpg)), full((1, cpg)), full((cpg, c)),
                  vec, vec, vec, vec, vec, vec, vec, vec],
        out_specs=pt_spec_c,
        out_shape=jax.ShapeDtypeStruct((n, c), jnp.float32),
    )(nv, spg, q_pts, nf0, W_d1, W_d2, b_d2[None, :], W_g, b_g[None, :], W_a1,
      W_a2, b_a2[None, :], emat, s1, t1, s2, t2, s3, t3, s4, t4)

    return out


# speedup vs baseline: 1.8376x; 1.0389x over previous
import functools

import jax
import jax.numpy as jnp
from jax import lax
from jax.experimental import pallas as pl
from jax.experimental.pallas import tpu as pltpu
from jax.experimental.pallas import tpu_sc as plsc

_NW = 32


def _sc_gather(feats, pts_lin, idx_flat):
    rows = idx_flat.shape[0]
    c = feats.shape[1]
    npts4 = pts_lin.shape[0]
    rpw = rows // _NW
    ch = 80 if rpw % 80 == 0 else rpw
    iters = rpw // ch
    nv16 = ch * 4 // 16
    mesh = plsc.VectorSubcoreMesh(core_axis_name="c", subcore_axis_name="s")

    @functools.partial(
        pl.kernel,
        mesh=mesh,
        compiler_params=pltpu.CompilerParams(needs_layout_passes=False),
        out_type=[
            jax.ShapeDtypeStruct((rows, c), jnp.float32),
            jax.ShapeDtypeStruct((rows * 4,), jnp.float32),
        ],
        scratch_types=[
            pltpu.VMEM((ch,), jnp.int32),
            pltpu.VMEM((ch, c), jnp.float32),
            pltpu.VMEM((npts4,), jnp.float32),
            pltpu.VMEM((ch * 4,), jnp.float32),
            pltpu.SemaphoreType.DMA,
        ],
    )
    def gk(idx_hbm, feats_hbm, ptslin_hbm, outf_hbm, outp_hbm,
           idx_v, rows_v, ptab_v, pbuf_v, sem1):
        wid = lax.axis_index("s") * 2 + lax.axis_index("c")
        base = wid * rpw
        pltpu.sync_copy(ptslin_hbm, ptab_v)
        io4 = lax.iota(jnp.int32, 16) // 4
        ioc = lax.iota(jnp.int32, 16) % 4

        def body(j, carry):
            off = base + j * ch
            pltpu.sync_copy(idx_hbm.at[pl.ds(off, ch)], idx_v)
            cp1 = pltpu.async_copy(feats_hbm.at[idx_v], rows_v, sem1)

            def inner(t, carry2):
                e0 = t * 16
                rvec = e0 // 4 + io4
                idxg = plsc.load_gather(idx_v, [rvec])
                vals = plsc.load_gather(ptab_v, [idxg * 4 + ioc])
                pbuf_v[pl.ds(e0, 16)] = vals
                return carry2

            lax.fori_loop(0, nv16, inner, 0)
            cp1.wait()
            pltpu.sync_copy(rows_v, outf_hbm.at[pl.ds(off, ch)])
            pltpu.sync_copy(pbuf_v, outp_hbm.at[pl.ds(off * 4, ch * 4)])
            return carry

        lax.fori_loop(0, iters, body, 0)

    outf, outp = gk(idx_flat, feats, pts_lin)
    return outf, outp.reshape(rows, 4)


def _lrelu(x):
    return jnp.where(x >= 0, x, 0.1 * x)


def _rep_points(v, bp, h):
    k = v.shape[1]
    return jnp.broadcast_to(v[:, None, :], (bp, h, k)).reshape(bp * h, k)


def _fused_body(nv_ref, spg_ref, qp_ref, wd1_ref, wd2_ref, bd2_ref,
                wdiag_ref, bg_ref, wa2_ref, ba2_ref, emat_ref,
                gd1_ref, bd1_ref, gg0_ref, bg0_ref, ga0_ref, ba0_ref,
                ga1_ref, ba1_ref,
                out_ref,
                mom, sums2, sums3, sums4,
                s1, t1, s2, t2, s3, t3, s4, t4,
                sge_s, sge2_s, nf0_s,
                *, rowsf, bp, h):
    p = pl.program_id(0)
    b = pl.program_id(1)
    br = bp * h
    c = nv_ref.shape[1]
    cpg = wa2_ref.shape[1]
    inv = 1.0 / rowsf

    def accum(ref, val):
        @pl.when(b == 0)
        def _():
            ref[...] = val

        @pl.when(b > 0)
        def _():
            ref[...] += val

    def finalize(sums_ref, gref, bref, sref, tref):
        mean = sums_ref[0:1, :] * inv
        var = sums_ref[1:2, :] * inv - mean * mean
        sv = gref[...] * lax.rsqrt(var + 1e-5)
        sref[...] = sv
        tref[...] = bref[...] - mean * sv

    @pl.when((p == 1) & (b == 0))
    def _():
        m0 = mom[...] * inv
        mean3 = m0[3:4, 0:3]
        e2 = m0[0:3, 0:3]
        a = wd1_ref[...]
        m1 = jnp.dot(mean3, a, preferred_element_type=jnp.float32)
        var1 = (jnp.sum(a * jnp.dot(e2, a, preferred_element_type=jnp.float32),
                        axis=0)[None, :] - m1 * m1)
        sv = gd1_ref[...] * lax.rsqrt(var1 + 1e-5)
        s1[...] = sv
        t1[...] = bd1_ref[...] - m1 * sv

    @pl.when((p == 2) & (b == 0))
    def _():
        finalize(sums2, gg0_ref, bg0_ref, s2, t2)

    @pl.when((p == 3) & (b == 0))
    def _():
        finalize(sums3, ga0_ref, ba0_ref, s3, t3)

    @pl.when((p == 4) & (b == 0))
    def _():
        finalize(sums4, ga1_ref, ba1_ref, s4, t4)

    def compute_ge():
        d3 = spg_ref[:, :3] - _rep_points(qp_ref[...], bp, h)
        y1 = jnp.dot(d3, wd1_ref[...], preferred_element_type=jnp.float32)
        a1 = _lrelu(y1 * s1[...] + t1[...])
        return (jnp.dot(a1, wd2_ref[...], preferred_element_type=jnp.float32)
                + bd2_ref[...])

    def q0_block():
        a = _lrelu(nf0_s[pl.ds(b * bp, bp), :] * s2[...] + t2[...])
        return (jnp.dot(a, wdiag_ref[0:c, 0:c], preferred_element_type=jnp.float32)
                + bg_ref[...])

    @pl.when(p == 0)
    def _():
        d3 = spg_ref[:, :3] - _rep_points(qp_ref[...], bp, h)
        d4 = jnp.concatenate([d3, jnp.ones((br, 1), jnp.float32)], axis=1)
        accum(mom, lax.dot_general(d4, d4, (((0,), (0,)), ((), ())),
                                   preferred_element_type=jnp.float32))

    @pl.when(p == 1)
    def _():
        ge = compute_ge()
        nvf = nv_ref[...] - ge
        ge3 = ge.reshape(bp, h, c)
        sge_s[pl.ds(b * bp, bp), :] = jnp.sum(ge3, axis=1)
        sge2_s[pl.ds(b * bp, bp), :] = jnp.sum(ge3 * ge3, axis=1)
        nf0_s[pl.ds(b * bp, bp), :] = nvf.reshape(bp, h, c)[:, 0, :]
        accum(sums2, jnp.concatenate([jnp.sum(nvf, axis=0)[None, :],
                                      jnp.sum(nvf * nvf, axis=0)[None, :]], axis=0))

    @pl.when(p == 2)
    def _():
        q0 = q0_block()
        sge = sge_s[pl.ds(b * bp, bp), :]
        sge2 = sge2_s[pl.ds(b * bp, bp), :]
        hf = jnp.float32(h)
        s = jnp.sum(hf * q0 - sge, axis=0)[None, :]
        q = jnp.sum(hf * (q0 * q0) - 2.0 * q0 * sge + sge2, axis=0)[None, :]
        accum(sums3, jnp.concatenate([s, q], axis=0))

    @pl.when(p == 3)
    def _():
        ge = compute_ge()
        qf = _rep_points(q0_block(), bp, h) - ge
        a3 = _lrelu(qf * s3[...] + t3[...])
        z4 = jnp.dot(a3, wdiag_ref[c:2 * c, c:2 * c],
                     preferred_element_type=jnp.float32)
        accum(sums4, jnp.concatenate([jnp.sum(z4, axis=0)[None, :],
                                      jnp.sum(z4 * z4, axis=0)[None, :]], axis=0))

    @pl.when(p == 4)
    def _():
        ge = compute_ge()
        nvf = nv_ref[...] - ge
        a2 = _lrelu(nvf * s2[...] + t2[...])
        qf = _rep_points(q0_block(), bp, h) - ge
        a3 = _lrelu(qf * s3[...] + t3[...])
        both = jnp.concatenate([a2, a3], axis=1)
        fused = jnp.dot(both, wdiag_ref[...], preferred_element_type=jnp.float32)
        nvf2 = fused[:, 0:c] + bg_ref[...]
        z4 = fused[:, c:2 * c]
        a4 = _lrelu(z4 * s4[...] + t4[...])
        aw = jnp.dot(a4, wa2_ref[...], preferred_element_type=jnp.float32) + ba2_ref[...]
        r = aw.reshape(bp, h, cpg)
        m = jnp.max(r, axis=1, keepdims=True)
        e = jnp.exp(r - m)
        sm = (e / jnp.sum(e, axis=1, keepdims=True)).reshape(br, cpg)
        smx = jnp.dot(sm, emat_ref[...], preferred_element_type=jnp.float32)
        out_ref[...] = jnp.sum((nvf2 * smx).reshape(bp, h, c), axis=1)


def kernel(q_pts, s_pts, s_feats, neighb_inds, W_d1, g_d1, b_d1, W_d2, b_d2,
           g_g0, b_g0, W_g, b_g, g_a0, b_a0, W_a1, g_a1, b_a1, W_a2, b_a2):
    n = q_pts.shape[0]
    h = neighb_inds.shape[1]
    c = s_feats.shape[1]
    cpg = W_a2.shape[1]
    g = c // cpg
    rows = n * h

    bp = 40 if n % 40 == 0 else 8
    br = bp * h
    nb = rows // br

    idx_flat = neighb_inds.reshape(-1).astype(jnp.int32)
    pts_lin = jnp.pad(s_pts, ((0, 0), (0, 4 - s_pts.shape[1]))).reshape(-1)

    nv, spg = _sc_gather(s_feats, pts_lin, idx_flat)

    zc = jnp.zeros((c, c), jnp.float32)
    wdiag = jnp.block([[W_g, zc], [zc, W_a1]])
    emat = (jnp.arange(c)[None, :] // g == jnp.arange(cpg)[:, None]).astype(jnp.float32)

    def rows_map(*phases):
        def f(p, b):
            sel = (p == phases[0])
            for q in phases[1:]:
                sel = sel | (p == q)
            return (jnp.where(sel, b, 0), 0)
        return f

    row_spec_c = pl.BlockSpec((br, c), rows_map(1, 4))
    row_spec_p = pl.BlockSpec((br, 4), rows_map(0, 1, 3, 4))
    qp_spec = pl.BlockSpec((bp, 3), rows_map(0, 1, 3, 4))
    full = lambda shape: pl.BlockSpec(shape, lambda p, b: tuple(0 for _ in shape))
    vec = full((1, c))

    out = pl.pallas_call(
        functools.partial(_fused_body, rowsf=float(rows), bp=bp, h=h),
        grid=(5, nb),
        in_specs=[row_spec_c, row_spec_p, qp_spec, full((3, c)), full((c, c)),
                  vec, full((2 * c, 2 * c)), vec, full((c, cpg)), full((1, cpg)),
                  full((cpg, c)), vec, vec, vec, vec, vec, vec, vec, vec],
        out_specs=pl.BlockSpec((bp, c), rows_map(4)),
        out_shape=jax.ShapeDtypeStruct((n, c), jnp.float32),
        scratch_shapes=[
            pltpu.VMEM((4, 4), jnp.float32),
            pltpu.VMEM((2, c), jnp.float32),
            pltpu.VMEM((2, c), jnp.float32),
            pltpu.VMEM((2, c), jnp.float32),
            pltpu.VMEM((1, c), jnp.float32),
            pltpu.VMEM((1, c), jnp.float32),
            pltpu.VMEM((1, c), jnp.float32),
            pltpu.VMEM((1, c), jnp.float32),
            pltpu.VMEM((1, c), jnp.float32),
            pltpu.VMEM((1, c), jnp.float32),
            pltpu.VMEM((1, c), jnp.float32),
            pltpu.VMEM((1, c), jnp.float32),
            pltpu.VMEM((n, c), jnp.float32),
            pltpu.VMEM((n, c), jnp.float32),
            pltpu.VMEM((n, c), jnp.float32),
        ],
    )(nv, spg, q_pts, W_d1, W_d2, b_d2[None, :], wdiag, b_g[None, :],
      W_a2, b_a2[None, :], emat, g_d1[None, :], b_d1[None, :],
      g_g0[None, :], b_g0[None, :], g_a0[None, :], b_a0[None, :],
      g_a1[None, :], b_a1[None, :])

    return out


# speedup vs baseline: 2.5680x; 1.3975x over previous
import functools

import jax
import jax.numpy as jnp
from jax import lax
from jax.experimental import pallas as pl
from jax.experimental.pallas import tpu as pltpu
from jax.experimental.pallas import tpu_sc as plsc

_NW = 32


def _sc_gather(feats, pts_lin, idx_flat):
    rows = idx_flat.shape[0]
    c = feats.shape[1]
    npts4 = pts_lin.shape[0]
    rpw = rows // _NW
    ch = 80 if rpw % 80 == 0 else rpw
    iters = rpw // ch
    nv16 = ch * 4 // 16
    mesh = plsc.VectorSubcoreMesh(core_axis_name="c", subcore_axis_name="s")

    @functools.partial(
        pl.kernel,
        mesh=mesh,
        compiler_params=pltpu.CompilerParams(needs_layout_passes=False),
        out_type=[
            jax.ShapeDtypeStruct((rows, c), jnp.float32),
            jax.ShapeDtypeStruct((rows * 4,), jnp.float32),
        ],
        scratch_types=[
            pltpu.VMEM((ch,), jnp.int32),
            pltpu.VMEM((ch, c), jnp.float32),
            pltpu.VMEM((npts4,), jnp.float32),
            pltpu.VMEM((ch * 4,), jnp.float32),
            pltpu.SemaphoreType.DMA,
        ],
    )
    def gk(idx_hbm, feats_hbm, ptslin_hbm, outf_hbm, outp_hbm,
           idx_v, rows_v, ptab_v, pbuf_v, sem1):
        wid = lax.axis_index("s") * 2 + lax.axis_index("c")
        base = wid * rpw
        pltpu.sync_copy(ptslin_hbm, ptab_v)
        io4 = lax.iota(jnp.int32, 16) // 4
        ioc = lax.iota(jnp.int32, 16) % 4

        def body(j, carry):
            off = base + j * ch
            pltpu.sync_copy(idx_hbm.at[pl.ds(off, ch)], idx_v)
            cp1 = pltpu.async_copy(feats_hbm.at[idx_v], rows_v, sem1)

            def inner(t, carry2):
                e0 = t * 16
                rvec = e0 // 4 + io4
                idxg = plsc.load_gather(idx_v, [rvec])
                vals = plsc.load_gather(ptab_v, [idxg * 4 + ioc])
                pbuf_v[pl.ds(e0, 16)] = vals
                return carry2

            lax.fori_loop(0, nv16, inner, 0)
            cp1.wait()
            pltpu.sync_copy(rows_v, outf_hbm.at[pl.ds(off, ch)])
            pltpu.sync_copy(pbuf_v, outp_hbm.at[pl.ds(off * 4, ch * 4)])
            return carry

        lax.fori_loop(0, iters, body, 0)

    outf, outp = gk(idx_flat, feats, pts_lin)
    return outf, outp.reshape(rows, 4)


def _lrelu(x):
    return jnp.where(x >= 0, x, 0.1 * x)


def _rep_points(v, bp, h):
    k = v.shape[1]
    return jnp.broadcast_to(v[:, None, :], (bp, h, k)).reshape(bp * h, k)


def _fused_body(nv_ref, spg_ref, qp_ref, wd1_ref, wd2_ref, bd2_ref,
                wdiag_ref, bg_ref, wa2_ref, ba2_ref, emat_ref,
                gd1_ref, bd1_ref, gg0_ref, bg0_ref, ga0_ref, ba0_ref,
                ga1_ref, ba1_ref,
                out_ref,
                mom, sums2, sums3, sums4,
                s1, t1, s2, t2, s3, t3, s4, t4,
                sge_s, sge2_s, nf0_s,
                *, rowsf, bp, h):
    p = pl.program_id(0)
    b = pl.program_id(1)
    br = bp * h
    c = nv_ref.shape[1]
    cpg = wa2_ref.shape[1]
    inv = 1.0 / rowsf

    def accum(ref, val):
        @pl.when(b == 0)
        def _():
            ref[...] = val

        @pl.when(b > 0)
        def _():
            ref[...] += val

    def finalize(sums_ref, gref, bref, sref, tref):
        mean = sums_ref[0:1, :] * inv
        var = sums_ref[1:2, :] * inv - mean * mean
        sv = gref[...] * lax.rsqrt(var + 1e-5)
        sref[...] = sv
        tref[...] = bref[...] - mean * sv

    @pl.when((p == 1) & (b == 0))
    def _():
        m0 = mom[...] * inv
        mean3 = m0[3:4, 0:3]
        e2 = m0[0:3, 0:3]
        a = wd1_ref[...]
        m1 = jnp.dot(mean3, a, preferred_element_type=jnp.float32)
        var1 = (jnp.sum(a * jnp.dot(e2, a, preferred_element_type=jnp.float32),
                        axis=0)[None, :] - m1 * m1)
        sv = gd1_ref[...] * lax.rsqrt(var1 + 1e-5)
        s1[...] = sv
        t1[...] = bd1_ref[...] - m1 * sv

    @pl.when((p == 2) & (b == 0))
    def _():
        finalize(sums2, gg0_ref, bg0_ref, s2, t2)

    @pl.when((p == 3) & (b == 0))
    def _():
        finalize(sums3, ga0_ref, ba0_ref, s3, t3)

    @pl.when((p == 4) & (b == 0))
    def _():
        finalize(sums4, ga1_ref, ba1_ref, s4, t4)

    def compute_ge():
        d3 = spg_ref[:, :3] - _rep_points(qp_ref[...], bp, h)
        y1 = jnp.dot(d3, wd1_ref[...], preferred_element_type=jnp.float32)
        a1 = _lrelu(y1 * s1[...] + t1[...])
        return (jnp.dot(a1, wd2_ref[...], preferred_element_type=jnp.float32)
                + bd2_ref[...])

    def q0_block():
        a = _lrelu(nf0_s[pl.ds(b * bp, bp), :] * s2[...] + t2[...])
        return (jnp.dot(a, wdiag_ref[0:c, 0:c], preferred_element_type=jnp.float32)
                + bg_ref[...])

    @pl.when(p == 0)
    def _():
        d3 = spg_ref[:, :3] - _rep_points(qp_ref[...], bp, h)
        d4 = jnp.concatenate([d3, jnp.ones((br, 1), jnp.float32)], axis=1)
        accum(mom, lax.dot_general(d4, d4, (((0,), (0,)), ((), ())),
                                   preferred_element_type=jnp.float32))

    @pl.when(p == 1)
    def _():
        ge = compute_ge()
        nvf = nv_ref[...] - ge
        ge3 = ge.reshape(bp, h, c)
        sge_s[pl.ds(b * bp, bp), :] = jnp.sum(ge3, axis=1)
        sge2_s[pl.ds(b * bp, bp), :] = jnp.sum(ge3 * ge3, axis=1)
        nf0_s[pl.ds(b * bp, bp), :] = nvf.reshape(bp, h, c)[:, 0, :]
        accum(sums2, jnp.concatenate([jnp.sum(nvf, axis=0)[None, :],
                                      jnp.sum(nvf * nvf, axis=0)[None, :]], axis=0))

    @pl.when(p == 2)
    def _():
        q0 = q0_block()
        sge = sge_s[pl.ds(b * bp, bp), :]
        sge2 = sge2_s[pl.ds(b * bp, bp), :]
        hf = jnp.float32(h)
        s = jnp.sum(hf * q0 - sge, axis=0)[None, :]
        q = jnp.sum(hf * (q0 * q0) - 2.0 * q0 * sge + sge2, axis=0)[None, :]
        accum(sums3, jnp.concatenate([s, q], axis=0))

    @pl.when(p == 3)
    def _():
        ge = compute_ge()
        qf = _rep_points(q0_block(), bp, h) - ge
        a3 = _lrelu(qf * s3[...] + t3[...])
        z4 = jnp.dot(a3, wdiag_ref[c:2 * c, c:2 * c],
                     preferred_element_type=jnp.float32)
        accum(sums4, jnp.concatenate([jnp.sum(z4, axis=0)[None, :],
                                      jnp.sum(z4 * z4, axis=0)[None, :]], axis=0))

    @pl.when(p == 4)
    def _():
        ge = compute_ge()
        nvf = nv_ref[...] - ge
        a2 = _lrelu(nvf * s2[...] + t2[...])
        qf = _rep_points(q0_block(), bp, h) - ge
        a3 = _lrelu(qf * s3[...] + t3[...])
        both = jnp.concatenate([a2, a3], axis=1)
        fused = jnp.dot(both, wdiag_ref[...], preferred_element_type=jnp.float32)
        nvf2 = fused[:, 0:c] + bg_ref[...]
        z4 = fused[:, c:2 * c]
        a4 = _lrelu(z4 * s4[...] + t4[...])
        aw = jnp.dot(a4, wa2_ref[...], preferred_element_type=jnp.float32) + ba2_ref[...]
        r = aw.reshape(bp, h, cpg)
        m = jnp.max(r, axis=1, keepdims=True)
        e = jnp.exp(r - m)
        sm = (e / jnp.sum(e, axis=1, keepdims=True)).reshape(br, cpg)
        smx = jnp.dot(sm, emat_ref[...], preferred_element_type=jnp.float32)
        out_ref[...] = jnp.sum((nvf2 * smx).reshape(bp, h, c), axis=1)


def kernel(q_pts, s_pts, s_feats, neighb_inds, W_d1, g_d1, b_d1, W_d2, b_d2,
           g_g0, b_g0, W_g, b_g, g_a0, b_a0, W_a1, g_a1, b_a1, W_a2, b_a2):
    n = q_pts.shape[0]
    h = neighb_inds.shape[1]
    c = s_feats.shape[1]
    cpg = W_a2.shape[1]
    g = c // cpg
    rows = n * h

    bp = 200 if n % 200 == 0 else 8
    br = bp * h
    nb = rows // br

    idx_flat = neighb_inds.reshape(-1).astype(jnp.int32)
    pts_lin = jnp.pad(s_pts, ((0, 0), (0, 4 - s_pts.shape[1]))).reshape(-1)

    nv, spg = _sc_gather(s_feats, pts_lin, idx_flat)

    zc = jnp.zeros((c, c), jnp.float32)
    wdiag = jnp.block([[W_g, zc], [zc, W_a1]])
    emat = (jnp.arange(c)[None, :] // g == jnp.arange(cpg)[:, None]).astype(jnp.float32)

    def rows_map(*phases):
        def f(p, b):
            sel = (p == phases[0])
            for q in phases[1:]:
                sel = sel | (p == q)
            return (jnp.where(sel, b, 0), 0)
        return f

    row_spec_c = pl.BlockSpec((br, c), rows_map(1, 4))
    row_spec_p = pl.BlockSpec((br, 4), rows_map(0, 1, 3, 4))
    qp_spec = pl.BlockSpec((bp, 3), rows_map(0, 1, 3, 4))
    full = lambda shape: pl.BlockSpec(shape, lambda p, b: tuple(0 for _ in shape))
    vec = full((1, c))

    out = pl.pallas_call(
        functools.partial(_fused_body, rowsf=float(rows), bp=bp, h=h),
        grid=(5, nb),
        in_specs=[row_spec_c, row_spec_p, qp_spec, full((3, c)), full((c, c)),
                  vec, full((2 * c, 2 * c)), vec, full((c, cpg)), full((1, cpg)),
                  full((cpg, c)), vec, vec, vec, vec, vec, vec, vec, vec],
        out_specs=pl.BlockSpec((bp, c), rows_map(4)),
        out_shape=jax.ShapeDtypeStruct((n, c), jnp.float32),
        scratch_shapes=[
            pltpu.VMEM((4, 4), jnp.float32),
            pltpu.VMEM((2, c), jnp.float32),
            pltpu.VMEM((2, c), jnp.float32),
            pltpu.VMEM((2, c), jnp.float32),
            pltpu.VMEM((1, c), jnp.float32),
            pltpu.VMEM((1, c), jnp.float32),
            pltpu.VMEM((1, c), jnp.float32),
            pltpu.VMEM((1, c), jnp.float32),
            pltpu.VMEM((1, c), jnp.float32),
            pltpu.VMEM((1, c), jnp.float32),
            pltpu.VMEM((1, c), jnp.float32),
            pltpu.VMEM((1, c), jnp.float32),
            pltpu.VMEM((n, c), jnp.float32),
            pltpu.VMEM((n, c), jnp.float32),
            pltpu.VMEM((n, c), jnp.float32),
        ],
    )(nv, spg, q_pts, W_d1, W_d2, b_d2[None, :], wdiag, b_g[None, :],
      W_a2, b_a2[None, :], emat, g_d1[None, :], b_d1[None, :],
      g_g0[None, :], b_g0[None, :], g_a0[None, :], b_a0[None, :],
      g_a1[None, :], b_a1[None, :])

    return out


# speedup vs baseline: 3.1695x; 1.2343x over previous
import functools

import jax
import jax.numpy as jnp
from jax import lax
from jax.experimental import pallas as pl
from jax.experimental.pallas import tpu as pltpu
from jax.experimental.pallas import tpu_sc as plsc

_NW = 32


def _sc_gather_pos(pts_lin, idx_flat):
    rows = idx_flat.shape[0]
    npts4 = pts_lin.shape[0]
    rpw = rows // _NW
    ch = 400 if rpw % 400 == 0 else (80 if rpw % 80 == 0 else rpw)
    iters = rpw // ch
    nv16 = ch * 4 // 16
    mesh = plsc.VectorSubcoreMesh(core_axis_name="c", subcore_axis_name="s")

    @functools.partial(
        pl.kernel,
        mesh=mesh,
        compiler_params=pltpu.CompilerParams(needs_layout_passes=False),
        out_type=jax.ShapeDtypeStruct((rows * 4,), jnp.float32),
        scratch_types=[
            pltpu.VMEM((ch,), jnp.int32),
            pltpu.VMEM((npts4,), jnp.float32),
            pltpu.VMEM((ch * 4,), jnp.float32),
        ],
    )
    def gk(idx_hbm, ptslin_hbm, outp_hbm, idx_v, ptab_v, pbuf_v):
        wid = lax.axis_index("s") * 2 + lax.axis_index("c")
        base = wid * rpw
        pltpu.sync_copy(ptslin_hbm, ptab_v)
        io4 = lax.iota(jnp.int32, 16) // 4
        ioc = lax.iota(jnp.int32, 16) % 4

        def body(j, carry):
            off = base + j * ch
            pltpu.sync_copy(idx_hbm.at[pl.ds(off, ch)], idx_v)

            def inner(t, carry2):
                e0 = t * 16
                rvec = e0 // 4 + io4
                idxg = plsc.load_gather(idx_v, [rvec])
                vals = plsc.load_gather(ptab_v, [idxg * 4 + ioc])
                pbuf_v[pl.ds(e0, 16)] = vals
                return carry2

            lax.fori_loop(0, nv16, inner, 0)
            pltpu.sync_copy(pbuf_v, outp_hbm.at[pl.ds(off * 4, ch * 4)])
            return carry

        lax.fori_loop(0, iters, body, 0)

    return gk(idx_flat, pts_lin).reshape(rows, 4)


def _sc_gather_feats(feats, idx_flat):
    rows = idx_flat.shape[0]
    c = feats.shape[1]
    rpw = rows // _NW
    ch = 80 if rpw % 80 == 0 else rpw
    iters = rpw // ch
    mesh = plsc.VectorSubcoreMesh(core_axis_name="c", subcore_axis_name="s")

    @functools.partial(
        pl.kernel,
        mesh=mesh,
        compiler_params=pltpu.CompilerParams(needs_layout_passes=False),
        out_type=jax.ShapeDtypeStruct((rows, c), jnp.float32),
        scratch_types=[
            pltpu.VMEM((ch,), jnp.int32),
            pltpu.VMEM((ch, c), jnp.float32),
            pltpu.SemaphoreType.DMA,
        ],
    )
    def gk(idx_hbm, feats_hbm, outf_hbm, idx_v, rows_v, sem1):
        wid = lax.axis_index("s") * 2 + lax.axis_index("c")
        base = wid * rpw

        def body(j, carry):
            off = base + j * ch
            pltpu.sync_copy(idx_hbm.at[pl.ds(off, ch)], idx_v)
            cp1 = pltpu.async_copy(feats_hbm.at[idx_v], rows_v, sem1)
            cp1.wait()
            pltpu.sync_copy(rows_v, outf_hbm.at[pl.ds(off, ch)])
            return carry

        lax.fori_loop(0, iters, body, 0)

    return gk(idx_flat, feats)


def _lrelu(x):
    return jnp.where(x >= 0, x, 0.1 * x)


def _rep_points(v, bp, h):
    k = v.shape[1]
    return jnp.broadcast_to(v[:, None, :], (bp, h, k)).reshape(bp * h, k)


def _ge_body(spg_ref, qp_ref, wd1_ref, wd2_ref, bd2_ref, gd1_ref, bd1_ref,
             ge_ref, sge_ref, sge2_ref,
             mom, s1, t1, *, rowsf, bp, h):
    p = pl.program_id(0)
    b = pl.program_id(1)
    br = bp * h
    c = wd2_ref.shape[1]
    inv = 1.0 / rowsf

    @pl.when(p == 0)
    def _():
        d3 = spg_ref[:, :3] - _rep_points(qp_ref[...], bp, h)
        d4 = jnp.concatenate([d3, jnp.ones((br, 1), jnp.float32)], axis=1)
        val = lax.dot_general(d4, d4, (((0,), (0,)), ((), ())),
                              preferred_element_type=jnp.float32)

        @pl.when(b == 0)
        def _():
            mom[...] = val

        @pl.when(b > 0)
        def _():
            mom[...] += val

    @pl.when((p == 1) & (b == 0))
    def _():
        m0 = mom[...] * inv
        mean3 = m0[3:4, 0:3]
        e2 = m0[0:3, 0:3]
        a = wd1_ref[...]
        m1 = jnp.dot(mean3, a, preferred_element_type=jnp.float32)
        var1 = (jnp.sum(a * jnp.dot(e2, a, preferred_element_type=jnp.float32),
                        axis=0)[None, :] - m1 * m1)
        sv = gd1_ref[...] * lax.rsqrt(var1 + 1e-5)
        s1[...] = sv
        t1[...] = bd1_ref[...] - m1 * sv

    @pl.when(p == 1)
    def _():
        d3 = spg_ref[:, :3] - _rep_points(qp_ref[...], bp, h)
        y1 = jnp.dot(d3, wd1_ref[...], preferred_element_type=jnp.float32)
        a1 = _lrelu(y1 * s1[...] + t1[...])
        ge = (jnp.dot(a1, wd2_ref[...], preferred_element_type=jnp.float32)
              + bd2_ref[...])
        ge_ref[...] = ge
        ge3 = ge.reshape(bp, h, c)
        sge_ref[...] = jnp.sum(ge3, axis=1)
        sge2_ref[...] = jnp.sum(ge3 * ge3, axis=1)


def _bn_body(nv_ref, ge_ref, sge_ref, sge2_ref, wg_ref, bg_ref, wa1_ref,
             gg0_ref, bg0_ref, ga0_ref, ba0_ref,
             z4_ref, st2_ref, sums4_ref,
             sums2, sums3, s2, t2, s3, t3, nf0_s, sums4_s,
             *, rowsf, bp, h, nb):
    p = pl.program_id(0)
    b = pl.program_id(1)
    c = nv_ref.shape[1]
    inv = 1.0 / rowsf

    def accum(ref, val):
        @pl.when(b == 0)
        def _():
            ref[...] = val

        @pl.when(b > 0)
        def _():
            ref[...] += val

    @pl.when(p == 0)
    def _():
        nvf = nv_ref[...] - ge_ref[...]
        nf0_s[pl.ds(b * bp, bp), :] = nvf.reshape(bp, h, c)[:, 0, :]
        accum(sums2, jnp.concatenate([jnp.sum(nvf, axis=0)[None, :],
                                      jnp.sum(nvf * nvf, axis=0)[None, :]], axis=0))

    @pl.when((p == 1) & (b == 0))
    def _():
        mean = sums2[0:1, :] * inv
        var = sums2[1:2, :] * inv - mean * mean
        sv = gg0_ref[...] * lax.rsqrt(var + 1e-5)
        s2[...] = sv
        t2[...] = bg0_ref[...] - mean * sv
        st2_ref[0:1, :] = sv
        st2_ref[1:2, :] = bg0_ref[...] - mean * sv

    def q0_block():
        a = _lrelu(nf0_s[pl.ds(b * bp, bp), :] * s2[...] + t2[...])
        return (jnp.dot(a, wg_ref[...], preferred_element_type=jnp.float32)
                + bg_ref[...])

    @pl.when(p == 1)
    def _():
        q0 = q0_block()
        sge = sge_ref[...]
        sge2 = sge2_ref[...]
        hf = jnp.float32(h)
        s = jnp.sum(hf * q0 - sge, axis=0)[None, :]
        q = jnp.sum(hf * (q0 * q0) - 2.0 * q0 * sge + sge2, axis=0)[None, :]
        accum(sums3, jnp.concatenate([s, q], axis=0))

    @pl.when((p == 2) & (b == 0))
    def _():
        mean = sums3[0:1, :] * inv
        var = sums3[1:2, :] * inv - mean * mean
        sv = ga0_ref[...] * lax.rsqrt(var + 1e-5)
        s3[...] = sv
        t3[...] = ba0_ref[...] - mean * sv

    @pl.when(p == 2)
    def _():
        qf = _rep_points(q0_block(), bp, h) - ge_ref[...]
        a3 = _lrelu(qf * s3[...] + t3[...])
        z4 = jnp.dot(a3, wa1_ref[...], preferred_element_type=jnp.float32)
        z4_ref[...] = z4
        accum(sums4_s, jnp.concatenate([jnp.sum(z4, axis=0)[None, :],
                                        jnp.sum(z4 * z4, axis=0)[None, :]], axis=0))

        @pl.when(b == nb - 1)
        def _():
            sums4_ref[...] = sums4_s[...]


def _final_body(nv_ref, ge_ref, z4_ref, st2_ref, sums4_ref,
                wg_ref, bg_ref, wa2_ref, ba2_ref, emat_ref,
                ga1_ref, ba1_ref,
                out_ref,
                s4, t4, *, rowsf, bp, h):
    b = pl.program_id(0)
    br = bp * h
    c = nv_ref.shape[1]
    cpg = wa2_ref.shape[1]
    inv = 1.0 / rowsf

    @pl.when(b == 0)
    def _():
        mean = sums4_ref[0:1, :] * inv
        var = sums4_ref[1:2, :] * inv - mean * mean
        sv = ga1_ref[...] * lax.rsqrt(var + 1e-5)
        s4[...] = sv
        t4[...] = ba1_ref[...] - mean * sv

    nvf = nv_ref[...] - ge_ref[...]
    a2 = _lrelu(nvf * st2_ref[0:1, :] + st2_ref[1:2, :])
    nvf2 = jnp.dot(a2, wg_ref[...], preferred_element_type=jnp.float32) + bg_ref[...]
    a4 = _lrelu(z4_ref[...] * s4[...] + t4[...])
    aw = jnp.dot(a4, wa2_ref[...], preferred_element_type=jnp.float32) + ba2_ref[...]
    r = aw.reshape(bp, h, cpg)
    m = jnp.max(r, axis=1, keepdims=True)
    e = jnp.exp(r - m)
    sm = (e / jnp.sum(e, axis=1, keepdims=True)).reshape(br, cpg)
    smx = jnp.dot(sm, emat_ref[...], preferred_element_type=jnp.float32)
    out_ref[...] = jnp.sum((nvf2 * smx).reshape(bp, h, c), axis=1)


def kernel(q_pts, s_pts, s_feats, neighb_inds, W_d1, g_d1, b_d1, W_d2, b_d2,
           g_g0, b_g0, W_g, b_g, g_a0, b_a0, W_a1, g_a1, b_a1, W_a2, b_a2):
    n = q_pts.shape[0]
    h = neighb_inds.shape[1]
    c = s_feats.shape[1]
    cpg = W_a2.shape[1]
    g = c // cpg
    rows = n * h

    bp = 200 if n % 200 == 0 else 8
    br = bp * h
    nb = rows // br

    idx_flat = neighb_inds.reshape(-1).astype(jnp.int32)
    pts_lin = jnp.pad(s_pts, ((0, 0), (0, 4 - s_pts.shape[1]))).reshape(-1)

    spg = _sc_gather_pos(pts_lin, idx_flat)
    nv = _sc_gather_feats(s_feats, idx_flat)

    emat = (jnp.arange(c)[None, :] // g == jnp.arange(cpg)[:, None]).astype(jnp.float32)

    def rows_map(*phases):
        def f(p, b):
            sel = (p == phases[0])
            for q in phases[1:]:
                sel = sel | (p == q)
            return (jnp.where(sel, b, 0), 0)
        return f

    full = lambda shape: pl.BlockSpec(shape, lambda *a: tuple(0 for _ in shape))
    vec = full((1, c))
    rowsf = float(rows)

    ge, sge, sge2 = pl.pallas_call(
        functools.partial(_ge_body, rowsf=rowsf, bp=bp, h=h),
        grid=(2, nb),
        in_specs=[pl.BlockSpec((br, 4), rows_map(0, 1)),
                  pl.BlockSpec((bp, 3), rows_map(0, 1)),
                  full((3, c)), full((c, c)), vec, vec, vec],
        out_specs=[pl.BlockSpec((br, c), rows_map(1)),
                   pl.BlockSpec((bp, c), rows_map(1)),
                   pl.BlockSpec((bp, c), rows_map(1))],
        out_shape=[jax.ShapeDtypeStruct((rows, c), jnp.float32),
                   jax.ShapeDtypeStruct((n, c), jnp.float32),
                   jax.ShapeDtypeStruct((n, c), jnp.float32)],
        scratch_shapes=[
            pltpu.VMEM((4, 4), jnp.float32),
            pltpu.VMEM((1, c), jnp.float32),
            pltpu.VMEM((1, c), jnp.float32),
        ],
    )(spg, q_pts, W_d1, W_d2, b_d2[None, :], g_d1[None, :], b_d1[None, :])

    z4, st2, sums4 = pl.pallas_call(
        functools.partial(_bn_body, rowsf=rowsf, bp=bp, h=h, nb=nb),
        grid=(3, nb),
        in_specs=[pl.BlockSpec((br, c), rows_map(0)),
                  pl.BlockSpec((br, c), rows_map(0, 2)),
                  pl.BlockSpec((bp, c), rows_map(1)),
                  pl.BlockSpec((bp, c), rows_map(1)),
                  full((c, c)), vec, full((c, c)),
                  vec, vec, vec, vec],
        out_specs=[pl.BlockSpec((br, c), rows_map(2)),
                   full((2, c)), full((2, c))],
        out_shape=[jax.ShapeDtypeStruct((rows, c), jnp.float32),
                   jax.ShapeDtypeStruct((2, c), jnp.float32),
                   jax.ShapeDtypeStruct((2, c), jnp.float32)],
        scratch_shapes=[
            pltpu.VMEM((2, c), jnp.float32),
            pltpu.VMEM((2, c), jnp.float32),
            pltpu.VMEM((1, c), jnp.float32),
            pltpu.VMEM((1, c), jnp.float32),
            pltpu.VMEM((1, c), jnp.float32),
            pltpu.VMEM((1, c), jnp.float32),
            pltpu.VMEM((n, c), jnp.float32),
            pltpu.VMEM((2, c), jnp.float32),
        ],
    )(nv, ge, sge, sge2, W_g, b_g[None, :], W_a1,
      g_g0[None, :], b_g0[None, :], g_a0[None, :], b_a0[None, :])

    blk = lambda shape: pl.BlockSpec(shape, lambda b: (b, 0))
    out = pl.pallas_call(
        functools.partial(_final_body, rowsf=rowsf, bp=bp, h=h),
        grid=(nb,),
        in_specs=[blk((br, c)), blk((br, c)), blk((br, c)),
                  full((2, c)), full((2, c)),
                  full((c, c)), vec, full((c, cpg)), full((1, cpg)),
                  full((cpg, c)), vec, vec],
        out_specs=blk((bp, c)),
        out_shape=jax.ShapeDtypeStruct((n, c), jnp.float32),
        scratch_shapes=[
            pltpu.VMEM((1, c), jnp.float32),
            pltpu.VMEM((1, c), jnp.float32),
        ],
    )(nv, ge, z4, st2, sums4, W_g, b_g[None, :], W_a2, b_a2[None, :],
      emat, g_a1[None, :], b_a1[None, :])

    return out


# speedup vs baseline: 3.1995x; 1.0095x over previous
import functools

import jax
import jax.numpy as jnp
from jax import lax
from jax.experimental import pallas as pl
from jax.experimental.pallas import tpu as pltpu
from jax.experimental.pallas import tpu_sc as plsc

_NW = 32


def _sc_gather_pos(pts_lin, idx_flat):
    rows = idx_flat.shape[0]
    npts4 = pts_lin.shape[0]
    rpw = rows // _NW
    ch = 400 if rpw % 400 == 0 else (80 if rpw % 80 == 0 else rpw)
    iters = rpw // ch
    nv16 = ch * 4 // 16
    mesh = plsc.VectorSubcoreMesh(core_axis_name="c", subcore_axis_name="s")

    @functools.partial(
        pl.kernel,
        mesh=mesh,
        compiler_params=pltpu.CompilerParams(needs_layout_passes=False),
        out_type=jax.ShapeDtypeStruct((rows * 4,), jnp.float32),
        scratch_types=[
            pltpu.VMEM((ch,), jnp.int32),
            pltpu.VMEM((npts4,), jnp.float32),
            pltpu.VMEM((ch * 4,), jnp.float32),
        ],
    )
    def gk(idx_hbm, ptslin_hbm, outp_hbm, idx_v, ptab_v, pbuf_v):
        wid = lax.axis_index("s") * 2 + lax.axis_index("c")
        base = wid * rpw
        pltpu.sync_copy(ptslin_hbm, ptab_v)
        io4 = lax.iota(jnp.int32, 16) // 4
        ioc = lax.iota(jnp.int32, 16) % 4

        def body(j, carry):
            off = base + j * ch
            pltpu.sync_copy(idx_hbm.at[pl.ds(off, ch)], idx_v)

            def inner(t, carry2):
                e0 = t * 16
                rvec = e0 // 4 + io4
                idxg = plsc.load_gather(idx_v, [rvec])
                vals = plsc.load_gather(ptab_v, [idxg * 4 + ioc])
                pbuf_v[pl.ds(e0, 16)] = vals
                return carry2

            lax.fori_loop(0, nv16, inner, 0)
            pltpu.sync_copy(pbuf_v, outp_hbm.at[pl.ds(off * 4, ch * 4)])
            return carry

        lax.fori_loop(0, iters, body, 0)

    return gk(idx_flat, pts_lin).reshape(rows, 4)


def _sc_gather_feats(feats, idx_flat):
    rows = idx_flat.shape[0]
    c = feats.shape[1]
    rpw = rows // _NW
    ch = 80 if rpw % 80 == 0 else rpw
    iters = rpw // ch
    mesh = plsc.VectorSubcoreMesh(core_axis_name="c", subcore_axis_name="s")

    pipelined = iters % 2 == 1 and iters >= 3
    pairs = (iters - 1) // 2

    @functools.partial(
        pl.kernel,
        mesh=mesh,
        compiler_params=pltpu.CompilerParams(needs_layout_passes=False),
        out_type=jax.ShapeDtypeStruct((rows, c), jnp.float32),
        scratch_types=[
            pltpu.VMEM((ch,), jnp.int32),
            pltpu.VMEM((ch, c), jnp.float32),
            pltpu.VMEM((ch,), jnp.int32),
            pltpu.VMEM((ch, c), jnp.float32),
            pltpu.SemaphoreType.DMA,
            pltpu.SemaphoreType.DMA,
        ],
    )
    def gk(idx_hbm, feats_hbm, outf_hbm, idx_a, rows_a, idx_b, rows_b,
           sem_a, sem_b):
        wid = lax.axis_index("s") * 2 + lax.axis_index("c")
        base = wid * rpw

        def start(j, idx_v, rows_v, sem):
            pltpu.sync_copy(idx_hbm.at[pl.ds(base + j * ch, ch)], idx_v)
            pltpu.async_copy(feats_hbm.at[idx_v], rows_v, sem)

        def finish(j, idx_v, rows_v, sem):
            pltpu.make_async_copy(feats_hbm.at[idx_v], rows_v, sem).wait()
            pltpu.sync_copy(rows_v, outf_hbm.at[pl.ds(base + j * ch, ch)])

        if pipelined:
            start(0, idx_a, rows_a, sem_a)

            def loop(k, carry):
                j = 2 * k
                start(j + 1, idx_b, rows_b, sem_b)
                finish(j, idx_a, rows_a, sem_a)
                start(j + 2, idx_a, rows_a, sem_a)
                finish(j + 1, idx_b, rows_b, sem_b)
                return carry

            lax.fori_loop(0, pairs, loop, 0)
            finish(iters - 1, idx_a, rows_a, sem_a)
        else:
            def loop(j, carry):
                start(j, idx_a, rows_a, sem_a)
                finish(j, idx_a, rows_a, sem_a)
                return carry

            lax.fori_loop(0, iters, loop, 0)

    return gk(idx_flat, feats)


def _lrelu(x):
    return jnp.where(x >= 0, x, 0.1 * x)


def _rep_points(v, bp, h):
    k = v.shape[1]
    return jnp.broadcast_to(v[:, None, :], (bp, h, k)).reshape(bp * h, k)


def _ge_body(spg_ref, qp_ref, wd1_ref, wd2_ref, bd2_ref, gd1_ref, bd1_ref,
             ge_ref, sge_ref, sge2_ref,
             mom, s1, t1, *, rowsf, bp, h):
    p = pl.program_id(0)
    b = pl.program_id(1)
    br = bp * h
    c = wd2_ref.shape[1]
    inv = 1.0 / rowsf

    @pl.when(p == 0)
    def _():
        d3 = spg_ref[:, :3] - _rep_points(qp_ref[...], bp, h)
        d4 = jnp.concatenate([d3, jnp.ones((br, 1), jnp.float32)], axis=1)
        val = lax.dot_general(d4, d4, (((0,), (0,)), ((), ())),
                              preferred_element_type=jnp.float32)

        @pl.when(b == 0)
        def _():
            mom[...] = val

        @pl.when(b > 0)
        def _():
            mom[...] += val

    @pl.when((p == 1) & (b == 0))
    def _():
        m0 = mom[...] * inv
        mean3 = m0[3:4, 0:3]
        e2 = m0[0:3, 0:3]
        a = wd1_ref[...]
        m1 = jnp.dot(mean3, a, preferred_element_type=jnp.float32)
        var1 = (jnp.sum(a * jnp.dot(e2, a, preferred_element_type=jnp.float32),
                        axis=0)[None, :] - m1 * m1)
        sv = gd1_ref[...] * lax.rsqrt(var1 + 1e-5)
        s1[...] = sv
        t1[...] = bd1_ref[...] - m1 * sv

    @pl.when(p == 1)
    def _():
        d3 = spg_ref[:, :3] - _rep_points(qp_ref[...], bp, h)
        y1 = jnp.dot(d3, wd1_ref[...], preferred_element_type=jnp.float32)
        a1 = _lrelu(y1 * s1[...] + t1[...])
        ge = (jnp.dot(a1, wd2_ref[...], preferred_element_type=jnp.float32)
              + bd2_ref[...])
        ge_ref[...] = ge
        ge3 = ge.reshape(bp, h, c)
        sge_ref[...] = jnp.sum(ge3, axis=1)
        sge2_ref[...] = jnp.sum(ge3 * ge3, axis=1)


def _bn_body(nv_ref, ge_ref, sge_ref, sge2_ref, wg_ref, bg_ref, wa1_ref,
             gg0_ref, bg0_ref, ga0_ref, ba0_ref,
             z4_ref, st2_ref, sums4_ref,
             sums2, sums3, s2, t2, s3, t3, nf0_s, sums4_s,
             *, rowsf, bp, h, nb):
    p = pl.program_id(0)
    b = pl.program_id(1)
    c = nv_ref.shape[1]
    inv = 1.0 / rowsf

    def accum(ref, val):
        @pl.when(b == 0)
        def _():
            ref[...] = val

        @pl.when(b > 0)
        def _():
            ref[...] += val

    @pl.when(p == 0)
    def _():
        nvf = nv_ref[...] - ge_ref[...]
        nf0_s[pl.ds(b * bp, bp), :] = nvf.reshape(bp, h, c)[:, 0, :]
        accum(sums2, jnp.concatenate([jnp.sum(nvf, axis=0)[None, :],
                                      jnp.sum(nvf * nvf, axis=0)[None, :]], axis=0))

    @pl.when((p == 1) & (b == 0))
    def _():
        mean = sums2[0:1, :] * inv
        var = sums2[1:2, :] * inv - mean * mean
        sv = gg0_ref[...] * lax.rsqrt(var + 1e-5)
        s2[...] = sv
        t2[...] = bg0_ref[...] - mean * sv
        st2_ref[0:1, :] = sv
        st2_ref[1:2, :] = bg0_ref[...] - mean * sv

    def q0_block():
        a = _lrelu(nf0_s[pl.ds(b * bp, bp), :] * s2[...] + t2[...])
        return (jnp.dot(a, wg_ref[...], preferred_element_type=jnp.float32)
                + bg_ref[...])

    @pl.when(p == 1)
    def _():
        q0 = q0_block()
        sge = sge_ref[...]
        sge2 = sge2_ref[...]
        hf = jnp.float32(h)
        s = jnp.sum(hf * q0 - sge, axis=0)[None, :]
        q = jnp.sum(hf * (q0 * q0) - 2.0 * q0 * sge + sge2, axis=0)[None, :]
        accum(sums3, jnp.concatenate([s, q], axis=0))

    @pl.when((p == 2) & (b == 0))
    def _():
        mean = sums3[0:1, :] * inv
        var = sums3[1:2, :] * inv - mean * mean
        sv = ga0_ref[...] * lax.rsqrt(var + 1e-5)
        s3[...] = sv
        t3[...] = ba0_ref[...] - mean * sv

    @pl.when(p == 2)
    def _():
        qf = _rep_points(q0_block(), bp, h) - ge_ref[...]
        a3 = _lrelu(qf * s3[...] + t3[...])
        z4 = jnp.dot(a3, wa1_ref[...], preferred_element_type=jnp.float32)
        z4_ref[...] = z4
        accum(sums4_s, jnp.concatenate([jnp.sum(z4, axis=0)[None, :],
                                        jnp.sum(z4 * z4, axis=0)[None, :]], axis=0))

        @pl.when(b == nb - 1)
        def _():
            sums4_ref[...] = sums4_s[...]


def _final_body(nv_ref, ge_ref, z4_ref, st2_ref, sums4_ref,
                wg_ref, bg_ref, wa2_ref, ba2_ref, emat_ref,
                ga1_ref, ba1_ref,
                out_ref,
                s4, t4, *, rowsf, bp, h):
    b = pl.program_id(0)
    br = bp * h
    c = nv_ref.shape[1]
    cpg = wa2_ref.shape[1]
    inv = 1.0 / rowsf

    @pl.when(b == 0)
    def _():
        mean = sums4_ref[0:1, :] * inv
        var = sums4_ref[1:2, :] * inv - mean * mean
        sv = ga1_ref[...] * lax.rsqrt(var + 1e-5)
        s4[...] = sv
        t4[...] = ba1_ref[...] - mean * sv

    nvf = nv_ref[...] - ge_ref[...]
    a2 = _lrelu(nvf * st2_ref[0:1, :] + st2_ref[1:2, :])
    nvf2 = jnp.dot(a2, wg_ref[...], preferred_element_type=jnp.float32) + bg_ref[...]
    a4 = _lrelu(z4_ref[...] * s4[...] + t4[...])
    aw = jnp.dot(a4, wa2_ref[...], preferred_element_type=jnp.float32) + ba2_ref[...]
    r = aw.reshape(bp, h, cpg)
    m = jnp.max(r, axis=1, keepdims=True)
    e = jnp.exp(r - m)
    sm = (e / jnp.sum(e, axis=1, keepdims=True)).reshape(br, cpg)
    smx = jnp.dot(sm, emat_ref[...], preferred_element_type=jnp.float32)
    out_ref[...] = jnp.sum((nvf2 * smx).reshape(bp, h, c), axis=1)


def kernel(q_pts, s_pts, s_feats, neighb_inds, W_d1, g_d1, b_d1, W_d2, b_d2,
           g_g0, b_g0, W_g, b_g, g_a0, b_a0, W_a1, g_a1, b_a1, W_a2, b_a2):
    n = q_pts.shape[0]
    h = neighb_inds.shape[1]
    c = s_feats.shape[1]
    cpg = W_a2.shape[1]
    g = c // cpg
    rows = n * h

    bp = 200 if n % 200 == 0 else 8
    br = bp * h
    nb = rows // br

    idx_flat = neighb_inds.reshape(-1).astype(jnp.int32)
    pts_lin = jnp.pad(s_pts, ((0, 0), (0, 4 - s_pts.shape[1]))).reshape(-1)

    spg = _sc_gather_pos(pts_lin, idx_flat)
    nv = _sc_gather_feats(s_feats, idx_flat)

    emat = (jnp.arange(c)[None, :] // g == jnp.arange(cpg)[:, None]).astype(jnp.float32)

    def rows_map(*phases):
        def f(p, b):
            sel = (p == phases[0])
            for q in phases[1:]:
                sel = sel | (p == q)
            return (jnp.where(sel, b, 0), 0)
        return f

    full = lambda shape: pl.BlockSpec(shape, lambda *a: tuple(0 for _ in shape))
    vec = full((1, c))
    rowsf = float(rows)

    ge, sge, sge2 = pl.pallas_call(
        functools.partial(_ge_body, rowsf=rowsf, bp=bp, h=h),
        grid=(2, nb),
        in_specs=[pl.BlockSpec((br, 4), rows_map(0, 1)),
                  pl.BlockSpec((bp, 3), rows_map(0, 1)),
                  full((3, c)), full((c, c)), vec, vec, vec],
        out_specs=[pl.BlockSpec((br, c), rows_map(1)),
                   pl.BlockSpec((bp, c), rows_map(1)),
                   pl.BlockSpec((bp, c), rows_map(1))],
        out_shape=[jax.ShapeDtypeStruct((rows, c), jnp.float32),
                   jax.ShapeDtypeStruct((n, c), jnp.float32),
                   jax.ShapeDtypeStruct((n, c), jnp.float32)],
        scratch_shapes=[
            pltpu.VMEM((4, 4), jnp.float32),
            pltpu.VMEM((1, c), jnp.float32),
            pltpu.VMEM((1, c), jnp.float32),
        ],
    )(spg, q_pts, W_d1, W_d2, b_d2[None, :], g_d1[None, :], b_d1[None, :])

    z4, st2, sums4 = pl.pallas_call(
        functools.partial(_bn_body, rowsf=rowsf, bp=bp, h=h, nb=nb),
        grid=(3, nb),
        in_specs=[pl.BlockSpec((br, c), rows_map(0)),
                  pl.BlockSpec((br, c), rows_map(0, 2)),
                  pl.BlockSpec((bp, c), rows_map(1)),
                  pl.BlockSpec((bp, c), rows_map(1)),
                  full((c, c)), vec, full((c, c)),
                  vec, vec, vec, vec],
        out_specs=[pl.BlockSpec((br, c), rows_map(2)),
                   full((2, c)), full((2, c))],
        out_shape=[jax.ShapeDtypeStruct((rows, c), jnp.float32),
                   jax.ShapeDtypeStruct((2, c), jnp.float32),
                   jax.ShapeDtypeStruct((2, c), jnp.float32)],
        scratch_shapes=[
            pltpu.VMEM((2, c), jnp.float32),
            pltpu.VMEM((2, c), jnp.float32),
            pltpu.VMEM((1, c), jnp.float32),
            pltpu.VMEM((1, c), jnp.float32),
            pltpu.VMEM((1, c), jnp.float32),
            pltpu.VMEM((1, c), jnp.float32),
            pltpu.VMEM((n, c), jnp.float32),
            pltpu.VMEM((2, c), jnp.float32),
        ],
    )(nv, ge, sge, sge2, W_g, b_g[None, :], W_a1,
      g_g0[None, :], b_g0[None, :], g_a0[None, :], b_a0[None, :])

    blk = lambda shape: pl.BlockSpec(shape, lambda b: (b, 0))
    out = pl.pallas_call(
        functools.partial(_final_body, rowsf=rowsf, bp=bp, h=h),
        grid=(nb,),
        in_specs=[blk((br, c)), blk((br, c)), blk((br, c)),
                  full((2, c)), full((2, c)),
                  full((c, c)), vec, full((c, cpg)), full((1, cpg)),
                  full((cpg, c)), vec, vec],
        out_specs=blk((bp, c)),
        out_shape=jax.ShapeDtypeStruct((n, c), jnp.float32),
        scratch_shapes=[
            pltpu.VMEM((1, c), jnp.float32),
            pltpu.VMEM((1, c), jnp.float32),
        ],
    )(nv, ge, z4, st2, sums4, W_g, b_g[None, :], W_a2, b_a2[None, :],
      emat, g_a1[None, :], b_a1[None, :])

    return out
